# baseline (speedup 1.0000x reference)
import functools

import jax
import jax.numpy as jnp
from jax import lax
from jax.experimental import pallas as pl
from jax.experimental.pallas import tpu as pltpu
from jax.experimental.pallas import tpu_sc as plsc

_NUM_CLASSES = 1000000
_HIDDEN = 64
_DROPOUT_PROB = 0.1
_BATCH = 16384

_NC = 2
_NS = 16
_NW = _NC * _NS
_BPW = _BATCH // _NW
_CHUNK = 128
_NCHUNK = _BPW // _CHUNK


@functools.partial(
    pl.kernel,
    mesh=plsc.VectorSubcoreMesh(core_axis_name="c", subcore_axis_name="s"),
    out_type=jax.ShapeDtypeStruct((_BATCH, _HIDDEN), jnp.float32),
    scratch_types=[
        pltpu.VMEM((_NCHUNK, _CHUNK), jnp.int32),
        pltpu.VMEM((_BPW, _HIDDEN), jnp.float32),
        pltpu.SemaphoreType.DMA,
    ],
    compiler_params=pltpu.CompilerParams(use_tc_tiling_on_sc=False),
)
def _gather_kernel(idx_hbm, table_hbm, out_hbm, idx_v, rows_v, sem):
    wid = lax.axis_index("s") * _NC + lax.axis_index("c")
    base = wid * _BPW
    pltpu.sync_copy(idx_hbm.at[wid], idx_v)
    copies = [
        pltpu.async_copy(
            table_hbm.at[idx_v.at[j]],
            rows_v.at[pl.ds(j * _CHUNK, _CHUNK)],
            sem,
        )
        for j in range(_NCHUNK)
    ]
    for c in copies:
        c.wait()
    pltpu.sync_copy(rows_v, out_hbm.at[pl.ds(base, _BPW)])


def kernel(labels, train, table):
    drop_ids = jax.random.uniform(jax.random.key(1), (labels.shape[0],)) < _DROPOUT_PROB
    labels = jnp.where((train != 0) & drop_ids, _NUM_CLASSES, labels.astype(jnp.int32))
    idx = labels.reshape(_NW, _NCHUNK, _CHUNK)
    return _gather_kernel(idx, table)


# speedup vs baseline: 1.6740x; 1.6740x over previous
import functools

import jax
import jax.numpy as jnp
from jax import lax
from jax.experimental import pallas as pl
from jax.experimental.pallas import tpu as pltpu
from jax.experimental.pallas import tpu_sc as plsc

_NUM_CLASSES = 1000000
_HIDDEN = 64
_DROPOUT_PROB = 0.1
_BATCH = 16384

_NC = 2
_NS = 16
_NW = _NC * _NS
_BPW = _BATCH // _NW
_K = 16
_NR = _BPW // _K


@functools.partial(
    pl.kernel,
    mesh=plsc.VectorSubcoreMesh(core_axis_name="c", subcore_axis_name="s"),
    out_type=jax.ShapeDtypeStruct((_BATCH, _HIDDEN), jnp.float32),
    scratch_types=[
        pltpu.VMEM((_BPW,), jnp.int32),
        pltpu.VMEM((_BPW, _HIDDEN), jnp.float32),
        pltpu.SemaphoreType.DMA,
    ],
)
def _gather_kernel(idx_hbm, table_hbm, out_hbm, idx_v, rows_v, rsem):
    wid = lax.axis_index("s") * _NC + lax.axis_index("c")
    base = wid * _BPW
    pltpu.sync_copy(idx_hbm.at[wid], idx_v)

    def issue_round(r):
        labs = idx_v[pl.ds(r * _K, _K)]
        for j in range(_K):
            pltpu.async_copy(table_hbm.at[labs[j]], rows_v.at[r * _K + j], rsem)

    def drain_round():
        pltpu.make_async_copy(
            table_hbm.at[pl.ds(0, _K)], rows_v.at[pl.ds(0, _K)], rsem
        ).wait()

    issue_round(0)

    def body(r, _):
        issue_round(r + 1)
        drain_round()
        return ()

    lax.fori_loop(0, _NR - 1, body, (), unroll=False)
    drain_round()
    pltpu.sync_copy(rows_v, out_hbm.at[pl.ds(base, _BPW)])


def kernel(labels, train, table):
    drop_ids = jax.random.uniform(jax.random.key(1), (labels.shape[0],)) < _DROPOUT_PROB
    labels = jnp.where((train != 0) & drop_ids, _NUM_CLASSES, labels.astype(jnp.int32))
    idx = labels.reshape(_NW, _BPW)
    return _gather_kernel(idx, table)
